# baseline (speedup 1.0000x reference)
import functools

import jax
import jax.numpy as jnp
from jax import lax
from jax.experimental import pallas as pl
from jax.experimental.pallas import tpu as pltpu
from jax.experimental.pallas import tpu_sc as plsc

NC = 2
NS = 16
L = 16
NW = NC * NS


def _sc_call(pts_T, gi_w, si_w, wpack, qpack):
    B = pts_T.shape[0]
    C = 3
    N = pts_T.shape[1] // C
    PW = si_w.shape[1]
    S = gi_w.shape[1] // PW
    WPB = NW // B
    NBLK = PW // L
    NWV = wpack.shape[0] // L

    mesh = plsc.VectorSubcoreMesh(
        core_axis_name="c", subcore_axis_name="s",
        num_cores=NC, num_subcores=NS)

    @functools.partial(
        pl.kernel,
        out_type=jax.ShapeDtypeStruct((NW, C * PW), jnp.float32),
        mesh=mesh,
        scratch_types=[
            pltpu.VMEM((C * N,), jnp.float32),
            pltpu.VMEM((S * PW,), jnp.int32),
            pltpu.VMEM((PW,), jnp.int32),
            pltpu.VMEM((wpack.shape[0],), jnp.float32),
            pltpu.VMEM((B * L,), jnp.float32),
            pltpu.VMEM((C * PW,), jnp.float32),
        ],
        compiler_params=pltpu.CompilerParams(needs_layout_passes=False),
    )
    def k(pts_hbm, gi_hbm, si_hbm, wpack_hbm, qpack_hbm, out_hbm,
          pts_v, gi_v, si_v, wpack_v, qpack_v, out_v):
        wid = lax.axis_index("s") * NC + lax.axis_index("c")
        b = wid // WPB
        pltpu.sync_copy(pts_hbm.at[b], pts_v)
        pltpu.sync_copy(gi_hbm.at[wid], gi_v)
        pltpu.sync_copy(si_hbm.at[wid], si_v)
        pltpu.sync_copy(wpack_hbm, wpack_v)
        pltpu.sync_copy(qpack_hbm, qpack_v)

        wvec = [wpack_v[pl.ds(i * L, L)] for i in range(NWV)]

        def wsc(k):
            return wvec[k // L][k % L]

        w1 = [[wsc(i * 8 + j) for j in range(8)] for i in range(3)]
        w2 = [[wsc(24 + i * 8 + j) for j in range(8)] for i in range(8)]
        w3 = [[wsc(88 + i * 16 + j) for j in range(16)] for i in range(8)]
        w4 = [[wsc(216 + i * 3 + j) for j in range(3)] for i in range(16)]
        qrow = qpack_v[pl.ds(pl.multiple_of(b * L, L), L)]
        qa = [qrow[i] for i in range(4)]
        qb = [qrow[4 + i] for i in range(4)]
        tt = [qrow[8 + i] for i in range(3)]

        one_v = jnp.full((L,), 1, jnp.int32)
        three_v = jnp.full((L,), 3, jnp.int32)
        lane_s = lax.iota(jnp.int32, L) * S

        def coords(idx):
            i3 = idx * three_v
            x = plsc.load_gather(pts_v, [i3])
            y = plsc.load_gather(pts_v, [i3 + one_v])
            z = plsc.load_gather(pts_v, [i3 + one_v + one_v])
            return x, y, z

        def mlp_step(nofs, cx, cy, cz, acc):
            nidx = plsc.load_gather(gi_v, [nofs + lane_s])
            gx, gy, gz = coords(nidx)
            rx = gx - cx
            ry = gy - cy
            rz = gz - cz
            h1 = [jnp.maximum(rx * w1[0][j] + ry * w1[1][j]
                              + rz * w1[2][j], 0.0)
                  for j in range(8)]
            h2 = []
            for j in range(8):
                v = h1[0] * w2[0][j]
                for i in range(1, 8):
                    v = v + h1[i] * w2[i][j]
                h2.append(jnp.maximum(v, 0.0))
            out = []
            for j in range(16):
                v = h2[0] * w3[0][j]
                for i in range(1, 8):
                    v = v + h2[i] * w3[i][j]
                out.append(jnp.maximum(acc[j], v))
            return tuple(out)

        UNROLL = 4

        def blk_body(blk, carry):
            pblk = pl.multiple_of(blk * L, L)
            cidx = si_v[pl.ds(pblk, L)]
            cx, cy, cz = coords(cidx)
            base = pblk * S

            def s_body(su, acc):
                s0 = su * UNROLL
                for u in range(UNROLL):
                    acc = mlp_step(base + s0 + u, cx, cy, cz, acc)
                return acc

            acc0 = tuple(jnp.zeros((L,), jnp.float32) for _ in range(16))
            feats = lax.fori_loop(0, S // UNROLL, s_body, acc0)

            flow = []
            for d in range(3):
                v = feats[0] * w4[0][d]
                for i in range(1, 16):
                    v = v + feats[i] * w4[i][d]
                flow.append(v)

            r0 = -(qa[1] * cx + qa[2] * cy + qa[3] * cz)
            r1 = qa[0] * cx - qa[2] * cz - qa[3] * cy
            r2 = qa[0] * cy - qa[1] * cz - qa[3] * cx
            r3 = qa[0] * cz - qa[1] * cy - qa[2] * cx
            wx = r0 * qb[1] - r1 * qb[0] - r2 * qb[3] - r3 * qb[2]
            wy = r0 * qb[2] - r1 * qb[3] - r2 * qb[0] - r3 * qb[1]
            wz = r0 * qb[3] - r1 * qb[2] - r2 * qb[1] - r3 * qb[0]

            out_v[pl.ds(pblk, L)] = wx + tt[0] + flow[0]
            out_v[pl.ds(pblk + PW, L)] = wy + tt[1] + flow[1]
            out_v[pl.ds(pblk + 2 * PW, L)] = wz + tt[2] + flow[2]
            return carry

        lax.fori_loop(0, NBLK, blk_body, 0)
        pltpu.sync_copy(out_v, out_hbm.at[wid])

    return k(pts_T, gi_w, si_w, wpack, qpack)


def kernel(points, q, t, sample_idx, group_idx, W1, W2, W3, W4):
    B, N, _ = points.shape
    P = sample_idx.shape[1]
    S = group_idx.shape[2]
    WPB = NW // B
    PW = P // WPB

    qf = jnp.reshape(q, (B, 4)).astype(jnp.float32)
    qn = qf / (jnp.sqrt(jnp.sum(qf * qf, axis=-1, keepdims=True) + 1e-10)
               + 1e-10)
    q2 = jnp.sum(qn * qn, axis=-1, keepdims=True) + 1e-10
    qinv = jnp.concatenate([qn[:, 0:1], -qn[:, 1:4]], axis=-1) / q2

    pts_T = points.astype(jnp.float32).reshape(B, N * 3)
    gi_w = group_idx.astype(jnp.int32).reshape(NW, PW * S)
    si_w = sample_idx.astype(jnp.int32).reshape(NW, PW)

    wpack = jnp.concatenate([
        W1.astype(jnp.float32).ravel(), W2.astype(jnp.float32).ravel(),
        W3.astype(jnp.float32).ravel(), W4.astype(jnp.float32).ravel(),
        jnp.zeros((8,), jnp.float32)])
    qpack = jnp.concatenate([
        qn, qinv, t.astype(jnp.float32),
        jnp.zeros((B, L - 11), jnp.float32)], axis=1).ravel()

    out = _sc_call(pts_T, gi_w, si_w, wpack, qpack)
    return (out.reshape(B, WPB, 3, PW)
            .transpose(0, 1, 3, 2)
            .reshape(B, P, 3))


# speedup vs baseline: 1.8385x; 1.8385x over previous
import functools

import jax
import jax.numpy as jnp
from jax import lax
from jax.experimental import pallas as pl
from jax.experimental.pallas import tpu as pltpu
from jax.experimental.pallas import tpu_sc as plsc

NC = 2
NS = 16
L = 16
NW = NC * NS


def _sc_call(pts_T, gi_w, si_w, wpack, qpack):
    B = pts_T.shape[0]
    C = 3
    N = pts_T.shape[1] // C
    PW = si_w.shape[1]
    S = gi_w.shape[1] // PW
    WPB = NW // B
    NBLK = PW // L
    NWV = wpack.shape[0] // L

    mesh = plsc.VectorSubcoreMesh(
        core_axis_name="c", subcore_axis_name="s",
        num_cores=NC, num_subcores=NS)

    @functools.partial(
        pl.kernel,
        out_type=jax.ShapeDtypeStruct((NW, C * PW), jnp.float32),
        mesh=mesh,
        scratch_types=[
            pltpu.VMEM((C * N,), jnp.float32),
            pltpu.VMEM((S * PW,), jnp.int32),
            pltpu.VMEM((PW,), jnp.int32),
            pltpu.VMEM((wpack.shape[0],), jnp.float32),
            pltpu.VMEM((B * L,), jnp.float32),
            pltpu.VMEM((C * PW,), jnp.float32),
            pltpu.VMEM((16 * L,), jnp.float32),
        ],
        compiler_params=pltpu.CompilerParams(needs_layout_passes=False),
    )
    def k(pts_hbm, gi_hbm, si_hbm, wpack_hbm, qpack_hbm, out_hbm,
          pts_v, gi_v, si_v, wpack_v, qpack_v, out_v, acc_v):
        wid = lax.axis_index("s") * NC + lax.axis_index("c")
        b = wid // WPB
        pltpu.sync_copy(pts_hbm.at[b], pts_v)
        pltpu.sync_copy(gi_hbm.at[wid], gi_v)
        pltpu.sync_copy(si_hbm.at[wid], si_v)
        pltpu.sync_copy(wpack_hbm, wpack_v)
        pltpu.sync_copy(qpack_hbm, qpack_v)

        wvec = [wpack_v[pl.ds(i * L, L)] for i in range(NWV)]

        def wsc(k):
            return wvec[k // L][k % L]

        w1 = [[wsc(i * 8 + j) for j in range(8)] for i in range(3)]
        w2 = [[wsc(24 + i * 8 + j) for j in range(8)] for i in range(8)]
        w3 = [[wsc(88 + i * 16 + j) for j in range(16)] for i in range(8)]
        w4 = [[wsc(216 + i * 3 + j) for j in range(3)] for i in range(16)]
        qrow = qpack_v[pl.ds(pl.multiple_of(b * L, L), L)]
        qa = [qrow[i] for i in range(4)]
        qb = [qrow[4 + i] for i in range(4)]
        tt = [qrow[8 + i] for i in range(3)]

        one_v = jnp.full((L,), 1, jnp.int32)
        three_v = jnp.full((L,), 3, jnp.int32)
        lane_s = lax.iota(jnp.int32, L) * S

        def coords(idx):
            i3 = idx * three_v
            x = plsc.load_gather(pts_v, [i3])
            y = plsc.load_gather(pts_v, [i3 + one_v])
            z = plsc.load_gather(pts_v, [i3 + one_v + one_v])
            return x, y, z

        def mlp_pair(nofs, cx, cy, cz, acc):
            nia = plsc.load_gather(gi_v, [nofs + lane_s])
            nib = plsc.load_gather(gi_v, [(nofs + 1) + lane_s])
            gxa, gya, gza = coords(nia)
            gxb, gyb, gzb = coords(nib)
            rxa, rya, rza = gxa - cx, gya - cy, gza - cz
            rxb, ryb, rzb = gxb - cx, gyb - cy, gzb - cz
            h1a, h1b = [], []
            for j in range(8):
                wx, wy, wz = w1[0][j], w1[1][j], w1[2][j]
                h1a.append(jnp.maximum(rxa * wx + rya * wy + rza * wz, 0.0))
                h1b.append(jnp.maximum(rxb * wx + ryb * wy + rzb * wz, 0.0))
            h2a, h2b = [], []
            for j in range(8):
                w = w2[0][j]
                va = h1a[0] * w
                vb = h1b[0] * w
                for i in range(1, 8):
                    w = w2[i][j]
                    va = va + h1a[i] * w
                    vb = vb + h1b[i] * w
                h2a.append(jnp.maximum(va, 0.0))
                h2b.append(jnp.maximum(vb, 0.0))
            out = []
            for j in range(16):
                w = w3[0][j]
                va = h2a[0] * w
                vb = h2b[0] * w
                for i in range(1, 8):
                    w = w3[i][j]
                    va = va + h2a[i] * w
                    vb = vb + h2b[i] * w
                out.append(jnp.maximum(acc[j], jnp.maximum(va, vb)))
            return tuple(out)

        def blk_body(blk, carry):
            pblk = pl.multiple_of(blk * L, L)
            cidx = si_v[pl.ds(pblk, L)]
            cx, cy, cz = coords(cidx)
            base = pblk * S

            def s_body(su, acc):
                return mlp_pair(base + su * 2, cx, cy, cz, acc)

            acc0 = tuple(jnp.zeros((L,), jnp.float32) for _ in range(16))
            feats = plsc.parallel_loop(0, S // 2, carry=acc0)(s_body)

            flow = []
            for d in range(3):
                v = feats[0] * w4[0][d]
                for i in range(1, 16):
                    v = v + feats[i] * w4[i][d]
                flow.append(v)

            r0 = -(qa[1] * cx + qa[2] * cy + qa[3] * cz)
            r1 = qa[0] * cx - qa[2] * cz - qa[3] * cy
            r2 = qa[0] * cy - qa[1] * cz - qa[3] * cx
            r3 = qa[0] * cz - qa[1] * cy - qa[2] * cx
            wx = r0 * qb[1] - r1 * qb[0] - r2 * qb[3] - r3 * qb[2]
            wy = r0 * qb[2] - r1 * qb[3] - r2 * qb[0] - r3 * qb[1]
            wz = r0 * qb[3] - r1 * qb[2] - r2 * qb[1] - r3 * qb[0]

            out_v[pl.ds(pblk, L)] = wx + tt[0] + flow[0]
            out_v[pl.ds(pblk + PW, L)] = wy + tt[1] + flow[1]
            out_v[pl.ds(pblk + 2 * PW, L)] = wz + tt[2] + flow[2]
            return carry

        lax.fori_loop(0, NBLK, blk_body, 0)
        pltpu.sync_copy(out_v, out_hbm.at[wid])

    return k(pts_T, gi_w, si_w, wpack, qpack)


def kernel(points, q, t, sample_idx, group_idx, W1, W2, W3, W4):
    B, N, _ = points.shape
    P = sample_idx.shape[1]
    S = group_idx.shape[2]
    WPB = NW // B
    PW = P // WPB

    qf = jnp.reshape(q, (B, 4)).astype(jnp.float32)
    qn = qf / (jnp.sqrt(jnp.sum(qf * qf, axis=-1, keepdims=True) + 1e-10)
               + 1e-10)
    q2 = jnp.sum(qn * qn, axis=-1, keepdims=True) + 1e-10
    qinv = jnp.concatenate([qn[:, 0:1], -qn[:, 1:4]], axis=-1) / q2

    pts_T = points.astype(jnp.float32).reshape(B, N * 3)
    gi_w = group_idx.astype(jnp.int32).reshape(NW, PW * S)
    si_w = sample_idx.astype(jnp.int32).reshape(NW, PW)

    wpack = jnp.concatenate([
        W1.astype(jnp.float32).ravel(), W2.astype(jnp.float32).ravel(),
        W3.astype(jnp.float32).ravel(), W4.astype(jnp.float32).ravel(),
        jnp.zeros((8,), jnp.float32)])
    qpack = jnp.concatenate([
        qn, qinv, t.astype(jnp.float32),
        jnp.zeros((B, L - 11), jnp.float32)], axis=1).ravel()

    out = _sc_call(pts_T, gi_w, si_w, wpack, qpack)
    return (out.reshape(B, WPB, 3, PW)
            .transpose(0, 1, 3, 2)
            .reshape(B, P, 3))


# speedup vs baseline: 2.1286x; 1.1578x over previous
import functools

import jax
import jax.numpy as jnp
from jax import lax
from jax.experimental import pallas as pl
from jax.experimental.pallas import tpu as pltpu
from jax.experimental.pallas import tpu_sc as plsc

NC = 2
NS = 16
L = 16
NW = NC * NS


def _sc_call(pts_T, gi_w, si_w, wpack, qpack):
    B = pts_T.shape[0]
    C = 3
    N = pts_T.shape[1] // C
    PW = si_w.shape[1]
    S = gi_w.shape[1] // PW
    WPB = NW // B
    NBLK = PW // L
    NWV = wpack.shape[0] // L

    mesh = plsc.VectorSubcoreMesh(
        core_axis_name="c", subcore_axis_name="s",
        num_cores=NC, num_subcores=NS)

    @functools.partial(
        pl.kernel,
        out_type=jax.ShapeDtypeStruct((NW, C * PW), jnp.float32),
        mesh=mesh,
        scratch_types=[
            pltpu.VMEM((C * N,), jnp.float32),
            pltpu.VMEM((S * PW,), jnp.int32),
            pltpu.VMEM((PW,), jnp.int32),
            pltpu.VMEM((wpack.shape[0],), jnp.float32),
            pltpu.VMEM((B * L,), jnp.float32),
            pltpu.VMEM((C * PW,), jnp.float32),
            pltpu.VMEM((16 * L,), jnp.float32),
        ],
        compiler_params=pltpu.CompilerParams(needs_layout_passes=False),
    )
    def k(pts_hbm, gi_hbm, si_hbm, wpack_hbm, qpack_hbm, out_hbm,
          pts_v, gi_v, si_v, wpack_v, qpack_v, out_v, acc_v):
        wid = lax.axis_index("s") * NC + lax.axis_index("c")
        b = wid // WPB
        pltpu.sync_copy(pts_hbm.at[b], pts_v)
        pltpu.sync_copy(gi_hbm.at[wid], gi_v)
        pltpu.sync_copy(si_hbm.at[wid], si_v)
        pltpu.sync_copy(wpack_hbm, wpack_v)
        pltpu.sync_copy(qpack_hbm, qpack_v)

        wvec = [wpack_v[pl.ds(i * L, L)] for i in range(NWV)]

        def wsc(k):
            return wvec[k // L][k % L]

        w1 = [[wsc(i * 8 + j) for j in range(8)] for i in range(3)]
        w2 = [[wsc(24 + i * 8 + j) for j in range(8)] for i in range(8)]
        w3 = [[wsc(88 + i * 16 + j) for j in range(16)] for i in range(8)]
        w4 = [[wsc(216 + i * 3 + j) for j in range(3)] for i in range(16)]
        qrow = qpack_v[pl.ds(pl.multiple_of(b * L, L), L)]
        qa = [qrow[i] for i in range(4)]
        qb = [qrow[4 + i] for i in range(4)]
        tt = [qrow[8 + i] for i in range(3)]

        one_v = jnp.full((L,), 1, jnp.int32)
        three_v = jnp.full((L,), 3, jnp.int32)
        lane_s = lax.iota(jnp.int32, L) * S

        def coords(idx):
            i3 = idx * three_v
            x = plsc.load_gather(pts_v, [i3])
            y = plsc.load_gather(pts_v, [i3 + one_v])
            z = plsc.load_gather(pts_v, [i3 + one_v + one_v])
            return x, y, z

        def mlp_step(nofs, cx, cy, cz, acc):
            nidx = plsc.load_gather(gi_v, [nofs + lane_s])
            gx, gy, gz = coords(nidx)
            rx = gx - cx
            ry = gy - cy
            rz = gz - cz
            h1 = [jnp.maximum(rx * w1[0][j] + ry * w1[1][j]
                              + rz * w1[2][j], 0.0)
                  for j in range(8)]
            h2 = []
            for j in range(8):
                v = h1[0] * w2[0][j]
                for i in range(1, 8):
                    v = v + h1[i] * w2[i][j]
                h2.append(jnp.maximum(v, 0.0))
            out = []
            for j in range(16):
                v = h2[0] * w3[0][j]
                for i in range(1, 8):
                    v = v + h2[i] * w3[i][j]
                out.append(jnp.maximum(acc[j], v))
            return tuple(out)

        def blk_body(blk, carry):
            pblk = pl.multiple_of(blk * L, L)
            cidx = si_v[pl.ds(pblk, L)]
            cx, cy, cz = coords(cidx)
            base = pblk * S

            def s_body(su, acc):
                return mlp_step(base + su, cx, cy, cz, acc)

            acc0 = tuple(jnp.zeros((L,), jnp.float32) for _ in range(16))
            feats = lax.fori_loop(0, S, s_body, acc0)

            flow = []
            for d in range(3):
                v = feats[0] * w4[0][d]
                for i in range(1, 16):
                    v = v + feats[i] * w4[i][d]
                flow.append(v)

            r0 = -(qa[1] * cx + qa[2] * cy + qa[3] * cz)
            r1 = qa[0] * cx - qa[2] * cz - qa[3] * cy
            r2 = qa[0] * cy - qa[1] * cz - qa[3] * cx
            r3 = qa[0] * cz - qa[1] * cy - qa[2] * cx
            wx = r0 * qb[1] - r1 * qb[0] - r2 * qb[3] - r3 * qb[2]
            wy = r0 * qb[2] - r1 * qb[3] - r2 * qb[0] - r3 * qb[1]
            wz = r0 * qb[3] - r1 * qb[2] - r2 * qb[1] - r3 * qb[0]

            out_v[pl.ds(pblk, L)] = wx + tt[0] + flow[0]
            out_v[pl.ds(pblk + PW, L)] = wy + tt[1] + flow[1]
            out_v[pl.ds(pblk + 2 * PW, L)] = wz + tt[2] + flow[2]
            return carry

        lax.fori_loop(0, NBLK, blk_body, 0)
        pltpu.sync_copy(out_v, out_hbm.at[wid])

    return k(pts_T, gi_w, si_w, wpack, qpack)


def kernel(points, q, t, sample_idx, group_idx, W1, W2, W3, W4):
    B, N, _ = points.shape
    P = sample_idx.shape[1]
    S = group_idx.shape[2]
    WPB = NW // B
    PW = P // WPB

    qf = jnp.reshape(q, (B, 4)).astype(jnp.float32)
    qn = qf / (jnp.sqrt(jnp.sum(qf * qf, axis=-1, keepdims=True) + 1e-10)
               + 1e-10)
    q2 = jnp.sum(qn * qn, axis=-1, keepdims=True) + 1e-10
    qinv = jnp.concatenate([qn[:, 0:1], -qn[:, 1:4]], axis=-1) / q2

    pts_T = points.astype(jnp.float32).reshape(B, N * 3)
    gi_w = group_idx.astype(jnp.int32).reshape(NW, PW * S)
    si_w = sample_idx.astype(jnp.int32).reshape(NW, PW)

    wpack = jnp.concatenate([
        W1.astype(jnp.float32).ravel(), W2.astype(jnp.float32).ravel(),
        W3.astype(jnp.float32).ravel(), W4.astype(jnp.float32).ravel(),
        jnp.zeros((8,), jnp.float32)])
    qpack = jnp.concatenate([
        qn, qinv, t.astype(jnp.float32),
        jnp.zeros((B, L - 11), jnp.float32)], axis=1).ravel()

    out = _sc_call(pts_T, gi_w, si_w, wpack, qpack)
    return (out.reshape(B, WPB, 3, PW)
            .transpose(0, 1, 3, 2)
            .reshape(B, P, 3))


# speedup vs baseline: 3.9649x; 1.8627x over previous
import functools

import jax
import jax.numpy as jnp
from jax import lax
from jax.experimental import pallas as pl
from jax.experimental.pallas import tpu as pltpu
from jax.experimental.pallas import tpu_sc as plsc

NC = 2
NS = 16
L = 16
NW = NC * NS


def _sc_call(pts_T, gi_w, si_w, wpack, qpack):
    B = pts_T.shape[0]
    C = 3
    N = pts_T.shape[1] // C
    PW = si_w.shape[1]
    S = gi_w.shape[1] // PW
    WPB = NW // B
    NBLK = PW // L
    NWV = wpack.shape[0] // L

    mesh = plsc.VectorSubcoreMesh(
        core_axis_name="c", subcore_axis_name="s",
        num_cores=NC, num_subcores=NS)

    @functools.partial(
        pl.kernel,
        out_type=jax.ShapeDtypeStruct((NW, C * PW), jnp.float32),
        mesh=mesh,
        scratch_types=[
            pltpu.VMEM((C * N,), jnp.float32),
            pltpu.VMEM((S * PW,), jnp.int32),
            pltpu.VMEM((PW,), jnp.int32),
            pltpu.VMEM((wpack.shape[0],), jnp.float32),
            pltpu.VMEM((B * L,), jnp.float32),
            pltpu.VMEM((C * PW,), jnp.float32),
            pltpu.VMEM((S * 8 * L,), jnp.float32),
            pltpu.VMEM((S * 8 * L,), jnp.float32),
        ],
        compiler_params=pltpu.CompilerParams(needs_layout_passes=False),
    )
    def k(pts_hbm, gi_hbm, si_hbm, wpack_hbm, qpack_hbm, out_hbm,
          pts_v, gi_v, si_v, wpack_v, qpack_v, out_v, h1_v, h2_v):
        wid = lax.axis_index("s") * NC + lax.axis_index("c")
        b = wid // WPB
        pltpu.sync_copy(pts_hbm.at[b], pts_v)
        pltpu.sync_copy(gi_hbm.at[wid], gi_v)
        pltpu.sync_copy(si_hbm.at[wid], si_v)
        pltpu.sync_copy(wpack_hbm, wpack_v)
        pltpu.sync_copy(qpack_hbm, qpack_v)

        wvec = [wpack_v[pl.ds(i * L, L)] for i in range(NWV)]

        def wsc(k):
            return wvec[k // L][k % L]

        w1 = [[wsc(i * 8 + j) for j in range(8)] for i in range(3)]
        w2 = [[wsc(24 + i * 8 + j) for j in range(8)] for i in range(8)]
        w3 = [[wsc(88 + i * 16 + j) for j in range(16)] for i in range(8)]
        w4 = [[wsc(216 + i * 3 + j) for j in range(3)] for i in range(16)]
        qrow = qpack_v[pl.ds(pl.multiple_of(b * L, L), L)]
        qa = [qrow[i] for i in range(4)]
        qb = [qrow[4 + i] for i in range(4)]
        tt = [qrow[8 + i] for i in range(3)]

        off_n = jnp.full((L,), N, jnp.int32)

        def coords(idx):
            idx_y = idx + off_n
            idx_z = idx_y + off_n
            x = plsc.load_gather(pts_v, [idx])
            y = plsc.load_gather(pts_v, [idx_y])
            z = plsc.load_gather(pts_v, [idx_z])
            return x, y, z

        zero_f = jnp.zeros((L,), jnp.float32)

        def blk_body(blk, carry):
            pblk = pl.multiple_of(blk * L, L)
            cidx = si_v[pl.ds(pblk, L)]
            cx, cy, cz = coords(cidx)

            def a_body(sp, c2):
                s2 = sp * 2
                nia = gi_v[pl.ds(pl.multiple_of(s2 * PW, L) + pblk, L)]
                nib = gi_v[pl.ds(pl.multiple_of((s2 + 1) * PW, L) + pblk, L)]
                gxa, gya, gza = coords(nia)
                gxb, gyb, gzb = coords(nib)
                rxa, rya, rza = gxa - cx, gya - cy, gza - cz
                rxb, ryb, rzb = gxb - cx, gyb - cy, gzb - cz
                soff = pl.multiple_of(s2 * (8 * L), 8 * L)
                for j in range(8):
                    wx, wy, wz = w1[0][j], w1[1][j], w1[2][j]
                    h1_v[pl.ds(soff + j * L, L)] = jnp.maximum(
                        rxa * wx + rya * wy + rza * wz, 0.0)
                    h1_v[pl.ds(soff + (8 + j) * L, L)] = jnp.maximum(
                        rxb * wx + ryb * wy + rzb * wz, 0.0)
                return c2
            lax.fori_loop(0, S // 2, a_body, 0)

            for half in range(2):
                cols = [[w2[i][half * 4 + j] for i in range(8)]
                        for j in range(4)]

                def b_body(s, c2, cols=cols, half=half):
                    soff = pl.multiple_of(s * (8 * L), 8 * L)
                    h1 = [h1_v[pl.ds(soff + i * L, L)] for i in range(8)]
                    for j in range(4):
                        v = h1[0] * cols[j][0]
                        for i in range(1, 8):
                            v = v + h1[i] * cols[j][i]
                        h2_v[pl.ds(soff + (half * 4 + j) * L, L)] = (
                            jnp.maximum(v, 0.0))
                    return c2
                lax.fori_loop(0, S, b_body, 0)

            fx, fy, fz = zero_f, zero_f, zero_f
            for q in range(4):
                cols = [[w3[i][q * 4 + j] for i in range(8)]
                        for j in range(4)]

                def c_body(s, acc4, cols=cols):
                    soff = pl.multiple_of(s * (8 * L), 8 * L)
                    h2 = [h2_v[pl.ds(soff + i * L, L)] for i in range(8)]
                    out4 = []
                    for j in range(4):
                        v = h2[0] * cols[j][0]
                        for i in range(1, 8):
                            v = v + h2[i] * cols[j][i]
                        out4.append(jnp.maximum(acc4[j], v))
                    return tuple(out4)
                acc4 = lax.fori_loop(0, S, c_body,
                                     (zero_f, zero_f, zero_f, zero_f))
                for j in range(4):
                    c = q * 4 + j
                    fx = fx + acc4[j] * w4[c][0]
                    fy = fy + acc4[j] * w4[c][1]
                    fz = fz + acc4[j] * w4[c][2]

            r0 = -(qa[1] * cx + qa[2] * cy + qa[3] * cz)
            r1 = qa[0] * cx - qa[2] * cz - qa[3] * cy
            r2 = qa[0] * cy - qa[1] * cz - qa[3] * cx
            r3 = qa[0] * cz - qa[1] * cy - qa[2] * cx
            wx = r0 * qb[1] - r1 * qb[0] - r2 * qb[3] - r3 * qb[2]
            wy = r0 * qb[2] - r1 * qb[3] - r2 * qb[0] - r3 * qb[1]
            wz = r0 * qb[3] - r1 * qb[2] - r2 * qb[1] - r3 * qb[0]

            out_v[pl.ds(pblk, L)] = wx + tt[0] + fx
            out_v[pl.ds(pblk + PW, L)] = wy + tt[1] + fy
            out_v[pl.ds(pblk + 2 * PW, L)] = wz + tt[2] + fz
            return carry

        lax.fori_loop(0, NBLK, blk_body, 0)
        pltpu.sync_copy(out_v, out_hbm.at[wid])

    return k(pts_T, gi_w, si_w, wpack, qpack)


def kernel(points, q, t, sample_idx, group_idx, W1, W2, W3, W4):
    B, N, _ = points.shape
    P = sample_idx.shape[1]
    S = group_idx.shape[2]
    WPB = NW // B
    PW = P // WPB

    qf = jnp.reshape(q, (B, 4)).astype(jnp.float32)
    qn = qf / (jnp.sqrt(jnp.sum(qf * qf, axis=-1, keepdims=True) + 1e-10)
               + 1e-10)
    q2 = jnp.sum(qn * qn, axis=-1, keepdims=True) + 1e-10
    qinv = jnp.concatenate([qn[:, 0:1], -qn[:, 1:4]], axis=-1) / q2

    pts_T = jnp.transpose(points.astype(jnp.float32),
                          (0, 2, 1)).reshape(B, 3 * N)
    gi_w = (jnp.transpose(group_idx.astype(jnp.int32), (0, 2, 1))
            .reshape(B, S, WPB, PW)
            .transpose(0, 2, 1, 3)
            .reshape(NW, S * PW))
    si_w = sample_idx.astype(jnp.int32).reshape(NW, PW)

    wpack = jnp.concatenate([
        W1.astype(jnp.float32).ravel(), W2.astype(jnp.float32).ravel(),
        W3.astype(jnp.float32).ravel(), W4.astype(jnp.float32).ravel(),
        jnp.zeros((8,), jnp.float32)])
    qpack = jnp.concatenate([
        qn, qinv, t.astype(jnp.float32),
        jnp.zeros((B, L - 11), jnp.float32)], axis=1).ravel()

    out = _sc_call(pts_T, gi_w, si_w, wpack, qpack)
    return (out.reshape(B, WPB, 3, PW)
            .transpose(0, 1, 3, 2)
            .reshape(B, P, 3))
